# baseline (speedup 1.0000x reference)
import functools
import math

import jax
import jax.numpy as jnp
import numpy as np
from jax import lax
from jax.experimental import pallas as pl
from jax.experimental.pallas import tpu as pltpu
from jax.experimental.pallas import tpu_sc as plsc

_NC = 2
_NS = 16
_NW = _NC * _NS
_LANES = 16
_CH = 16
_NG = 4
_NST = 2
_KB = 8


def _pe_table(seq_len: int, dim: int) -> np.ndarray:
    position = np.arange(0, seq_len, dtype=np.float32)[:, None]
    div_term = np.exp(
        np.arange(0, dim, 2, dtype=np.float32) * -(math.log(10000.0) / dim))
    pe = np.zeros((seq_len, dim), dtype=np.float32)
    pe[:, 0::2] = np.sin(position * div_term)
    pe[:, 1::2] = np.cos(position * div_term)
    return pe


def kernel(src, table):
    seq_len, batch = src.shape
    vocab, dim = table.shape
    rows = seq_len * batch
    rows_w = rows // _NW
    nch = rows_w // _CH
    pe_w = rows_w // batch
    ch_per_pos = batch // _CH
    nkb = dim // (_LANES * _KB)
    step = _NG * _NST // math.gcd(_NG, _NST)
    assert rows % _NW == 0 and rows_w % _CH == 0 and batch % _CH == 0
    assert dim % (_LANES * _KB) == 0 and nch >= 2 * step

    scale = 1.0 + math.sqrt(float(batch))
    pe = jnp.asarray(_pe_table(seq_len, dim))
    src_r = src.reshape(_NW, nch, _CH).astype(jnp.int32)

    mesh = plsc.VectorSubcoreMesh(core_axis_name="c", subcore_axis_name="s")

    @functools.partial(
        pl.kernel,
        out_type=jax.ShapeDtypeStruct((rows, dim), jnp.float32),
        mesh=mesh,
        scratch_types=[
            pltpu.VMEM((nch, _CH), jnp.int32),
            pltpu.VMEM((pe_w, dim), jnp.float32),
            [pltpu.VMEM((_CH, dim), jnp.float32)] * _NG,
            [pltpu.VMEM((_CH, dim), jnp.float32)] * _NST,
            [pltpu.SemaphoreType.DMA] * _NG,
            [pltpu.SemaphoreType.DMA] * _NST,
        ],
    )
    def emb_kernel(src_hbm, pe_hbm, table_hbm, out_hbm,
                   idx_v, pe_v, gbufs, sbufs, gsems, ssems):
        wid = lax.axis_index("s") * _NC + lax.axis_index("c")
        row_base = wid * rows_w
        pltpu.sync_copy(src_hbm.at[wid], idx_v)
        pltpu.sync_copy(pe_hbm.at[pl.ds(wid * pe_w, pe_w)], pe_v)

        def out_at(cc):
            return out_hbm.at[pl.ds(row_base + cc * _CH, _CH)]

        def fma(p, gbuf, sbuf):
            for kb in range(nkb):
                pvs = [pe_v[p, pl.ds((kb * _KB + k2) * _LANES, _LANES)]
                       for k2 in range(_KB)]

                @pl.loop(0, _CH, unroll=2)
                def _row(j):
                    for k2 in range(_KB):
                        sl = pl.ds((kb * _KB + k2) * _LANES, _LANES)
                        sbuf[j, sl] = gbuf[j, sl] * scale + pvs[k2]

        for b in range(_NG):
            pltpu.async_copy(table_hbm.at[idx_v.at[b]], gbufs[b], gsems[b])

        def body(cc, bg, bs, is_static):
            gbuf, sbuf = gbufs[bg], sbufs[bs]
            gsem, ssem = gsems[bg], ssems[bs]
            pltpu.make_async_copy(table_hbm.at[idx_v.at[cc]],
                                  gbuf, gsem).wait()

            def _drain():
                pltpu.make_async_copy(sbuf, out_at(cc), ssem).wait()

            if is_static:
                if cc >= _NST:
                    _drain()
            else:
                pl.when(cc >= _NST)(_drain)

            fma(cc // ch_per_pos, gbuf, sbuf)
            pltpu.async_copy(sbuf, out_at(cc), ssem)

            def _refill():
                pltpu.async_copy(table_hbm.at[idx_v.at[cc + _NG]],
                                 gbuf, gsem)

            if is_static:
                if cc + _NG < nch:
                    _refill()
            else:
                pl.when(cc + _NG < nch)(_refill)

        nmain = (nch // step) * step

        @pl.loop(0, nmain, step=step)
        def _chunk(c):
            for b in range(step):
                body(c + b, b % _NG, b % _NST, False)

        for cc in range(nmain, nch):
            body(cc, cc % _NG, cc % _NST, True)

        for cc in range(nch - _NST, nch):
            pltpu.make_async_copy(sbufs[cc % _NST], out_at(cc),
                                  ssems[cc % _NST]).wait()

    out = emb_kernel(src_r, pe, table)
    return out.reshape(seq_len, batch, dim)


# speedup vs baseline: 1.3353x; 1.3353x over previous
import functools
import math

import jax
import jax.numpy as jnp
import numpy as np
from jax import lax
from jax.experimental import pallas as pl
from jax.experimental.pallas import tpu as pltpu
from jax.experimental.pallas import tpu_sc as plsc

_NC = 2
_NS = 16
_NW = _NC * _NS
_LANES = 16
_CH = 16
_NG = 4
_NST = 2
_KB = 8


def _pe_table(seq_len: int, dim: int) -> np.ndarray:
    position = np.arange(0, seq_len, dtype=np.float32)[:, None]
    div_term = np.exp(
        np.arange(0, dim, 2, dtype=np.float32) * -(math.log(10000.0) / dim))
    pe = np.zeros((seq_len, dim), dtype=np.float32)
    pe[:, 0::2] = np.sin(position * div_term)
    pe[:, 1::2] = np.cos(position * div_term)
    return pe


def kernel(src, table):
    seq_len, batch = src.shape
    vocab, dim = table.shape
    rows = seq_len * batch
    rows_w = rows // _NW
    nch = rows_w // _CH
    pe_w = rows_w // batch
    ch_per_pos = batch // _CH
    nkb = dim // (_LANES * _KB)
    step = _NG * _NST // math.gcd(_NG, _NST)
    assert rows % _NW == 0 and rows_w % _CH == 0 and batch % _CH == 0
    assert dim % (_LANES * _KB) == 0 and nch >= 2 * step

    scale = 1.0 + math.sqrt(float(batch))
    pe = jnp.asarray(_pe_table(seq_len, dim))
    src_r = src.reshape(_NW, nch, _CH).astype(jnp.int32)

    mesh = plsc.VectorSubcoreMesh(core_axis_name="c", subcore_axis_name="s")

    @functools.partial(
        pl.kernel,
        out_type=jax.ShapeDtypeStruct((rows, dim), jnp.float32),
        mesh=mesh,
        scratch_types=[
            pltpu.VMEM((nch, _CH), jnp.int32),
            pltpu.VMEM((pe_w, dim), jnp.float32),
            [pltpu.VMEM((_CH, dim), jnp.float32)] * _NG,
            [pltpu.VMEM((_CH, dim), jnp.float32)] * _NST,
            [pltpu.SemaphoreType.DMA] * _NG,
            [pltpu.SemaphoreType.DMA] * _NST,
        ],
    )
    def emb_kernel(src_hbm, pe_hbm, table_hbm, out_hbm,
                   idx_v, pe_v, gbufs, sbufs, gsems, ssems):
        wid = lax.axis_index("c") * _NS + lax.axis_index("s")
        row_base = wid * rows_w
        pltpu.sync_copy(src_hbm.at[wid], idx_v)
        pltpu.sync_copy(pe_hbm.at[pl.ds(wid * pe_w, pe_w)], pe_v)

        def out_at(cc):
            return out_hbm.at[pl.ds(row_base + cc * _CH, _CH)]

        def fma(p, gbuf, sbuf):
            for kb in range(nkb):
                pvs = [pe_v[p, pl.ds((kb * _KB + k2) * _LANES, _LANES)]
                       for k2 in range(_KB)]

                @pl.loop(0, _CH)
                def _row(j):
                    for k2 in range(_KB):
                        sl = pl.ds((kb * _KB + k2) * _LANES, _LANES)
                        sbuf[j, sl] = gbuf[j, sl] * scale + pvs[k2]

        for b in range(_NG):
            pltpu.async_copy(table_hbm.at[idx_v.at[b]], gbufs[b], gsems[b])

        def body(cc, bg, bs, is_static):
            gbuf, sbuf = gbufs[bg], sbufs[bs]
            gsem, ssem = gsems[bg], ssems[bs]
            pltpu.make_async_copy(table_hbm.at[idx_v.at[cc]],
                                  gbuf, gsem).wait()

            def _drain():
                pltpu.make_async_copy(sbuf, out_at(cc), ssem).wait()

            if is_static:
                if cc >= _NST:
                    _drain()
            else:
                pl.when(cc >= _NST)(_drain)

            fma(cc // ch_per_pos, gbuf, sbuf)
            pltpu.async_copy(sbuf, out_at(cc), ssem)

            def _refill():
                pltpu.async_copy(table_hbm.at[idx_v.at[cc + _NG]],
                                 gbuf, gsem)

            if is_static:
                if cc + _NG < nch:
                    _refill()
            else:
                pl.when(cc + _NG < nch)(_refill)

        nmain = (nch // step) * step

        @pl.loop(0, nmain, step=step)
        def _chunk(c):
            for b in range(step):
                body(c + b, b % _NG, b % _NST, False)

        for cc in range(nmain, nch):
            body(cc, cc % _NG, cc % _NST, True)

        for cc in range(nch - _NST, nch):
            pltpu.make_async_copy(sbufs[cc % _NST], out_at(cc),
                                  ssems[cc % _NST]).wait()

    out = emb_kernel(src_r, pe, table)
    return out.reshape(seq_len, batch, dim)


# speedup vs baseline: 1.3417x; 1.0049x over previous
import functools
import math

import jax
import jax.numpy as jnp
import numpy as np
from jax import lax
from jax.experimental import pallas as pl
from jax.experimental.pallas import tpu as pltpu
from jax.experimental.pallas import tpu_sc as plsc

_NC = 2
_NS = 16
_NW = _NC * _NS
_LANES = 16
_CH = 16
_NG = 4
_NST = 2
_KB = 8


def _pe_table(seq_len: int, dim: int) -> np.ndarray:
    position = np.arange(0, seq_len, dtype=np.float32)[:, None]
    div_term = np.exp(
        np.arange(0, dim, 2, dtype=np.float32) * -(math.log(10000.0) / dim))
    pe = np.zeros((seq_len, dim), dtype=np.float32)
    pe[:, 0::2] = np.sin(position * div_term)
    pe[:, 1::2] = np.cos(position * div_term)
    return pe


def kernel(src, table):
    seq_len, batch = src.shape
    vocab, dim = table.shape
    rows = seq_len * batch
    rows_w = rows // _NW
    nch = rows_w // _CH
    pe_w = rows_w // batch
    ch_per_pos = batch // _CH
    nkb = dim // (_LANES * _KB)
    step = _NG * _NST // math.gcd(_NG, _NST)
    assert rows % _NW == 0 and rows_w % _CH == 0 and batch % _CH == 0
    assert dim % (_LANES * _KB) == 0 and nch >= 2 * step

    scale = 1.0 + math.sqrt(float(batch))
    pe = jnp.asarray(_pe_table(seq_len, dim))
    src_r = src.reshape(_NW, nch, _CH).astype(jnp.int32)

    mesh = plsc.VectorSubcoreMesh(core_axis_name="c", subcore_axis_name="s")

    @functools.partial(
        pl.kernel,
        out_type=jax.ShapeDtypeStruct((rows, dim), jnp.float32),
        mesh=mesh,
        scratch_types=[
            pltpu.VMEM((nch, _CH), jnp.int32),
            pltpu.VMEM((pe_w, dim), jnp.float32),
            [pltpu.VMEM((_CH, dim), jnp.float32)] * _NG,
            [pltpu.VMEM((_CH, dim), jnp.float32)] * _NST,
            [pltpu.SemaphoreType.DMA] * _NG,
            [pltpu.SemaphoreType.DMA] * _NST,
        ],
    )
    def emb_kernel(src_hbm, pe_hbm, table_hbm, out_hbm,
                   idx_v, pe_v, gbufs, sbufs, gsems, ssems):
        wid = lax.axis_index("c") * _NS + lax.axis_index("s")
        row_base = wid * rows_w
        pltpu.sync_copy(src_hbm.at[wid], idx_v)

        def out_at(cc):
            return out_hbm.at[pl.ds(row_base + cc * _CH, _CH)]

        def fma(p, gbuf, sbuf):
            for kb in range(nkb):
                pvs = [pe_v[p, pl.ds((kb * _KB + k2) * _LANES, _LANES)]
                       for k2 in range(_KB)]

                @pl.loop(0, _CH)
                def _row(j):
                    for k2 in range(_KB):
                        sl = pl.ds((kb * _KB + k2) * _LANES, _LANES)
                        sbuf[j, sl] = gbuf[j, sl] * scale + pvs[k2]

        for b in range(_NG):
            pltpu.async_copy(table_hbm.at[idx_v.at[b]], gbufs[b], gsems[b])
        pltpu.sync_copy(pe_hbm.at[pl.ds(wid * pe_w, pe_w)], pe_v)

        def body(cc, bg, bs, is_static):
            gbuf, sbuf = gbufs[bg], sbufs[bs]
            gsem, ssem = gsems[bg], ssems[bs]
            pltpu.make_async_copy(table_hbm.at[idx_v.at[cc]],
                                  gbuf, gsem).wait()

            def _drain():
                pltpu.make_async_copy(sbuf, out_at(cc), ssem).wait()

            if is_static:
                if cc >= _NST:
                    _drain()
            else:
                pl.when(cc >= _NST)(_drain)

            bp = (bg + _NG - 1) % _NG

            def _refill():
                pltpu.async_copy(table_hbm.at[idx_v.at[cc + _NG - 1]],
                                 gbufs[bp], gsems[bp])

            if is_static:
                if 1 <= cc and cc + _NG - 1 < nch:
                    _refill()
            else:
                pl.when(jnp.logical_and(cc >= 1, cc + _NG - 1 < nch))(_refill)

            fma(cc // ch_per_pos, gbuf, sbuf)
            pltpu.async_copy(sbuf, out_at(cc), ssem)

        nmain = (nch // step) * step

        @pl.loop(0, nmain, step=step)
        def _chunk(c):
            for b in range(step):
                body(c + b, b % _NG, b % _NST, False)

        for cc in range(nmain, nch):
            body(cc, cc % _NG, cc % _NST, True)

        for cc in range(nch - _NST, nch):
            pltpu.make_async_copy(sbufs[cc % _NST], out_at(cc),
                                  ssems[cc % _NST]).wait()

    out = emb_kernel(src_r, pe, table)
    return out.reshape(seq_len, batch, dim)
